# Initial kernel scaffold; baseline (speedup 1.0000x reference)
import jax
import jax.numpy as jnp
from jax.experimental import pallas as pl


def kernel(x, edge_index, batch, W1, b1, W2, b2, fc_W, fc_b):
    raise NotImplementedError("write your pallas kernel here")


# speedup vs baseline: 17.2747x; 17.2747x over previous
import functools

import jax
import jax.numpy as jnp
from jax import lax
from jax.experimental import pallas as pl
from jax.experimental.pallas import tpu as pltpu
from jax.experimental.pallas import tpu_sc as plsc

N = 10000
E = 320000
D = 128
G = 64

NC = 2
NS = 16
NW = NC * NS
EB = 80
ET = E // NW
NBT = ET // EB
NRA = 632
NRL = N - (NS - 1) * NRA
BR = 400
NBR = N // BR


def _sc_mesh():
    return plsc.VectorSubcoreMesh(core_axis_name="c", subcore_axis_name="s")


def _per_tile_rows(s, mk):
    @pl.when(s < NS - 1)
    def _():
        mk(pl.multiple_of(s * NRA, 8), NRA)

    @pl.when(s == NS - 1)
    def _():
        mk((NS - 1) * NRA, NRL)


def _sc_degree(dst2d, ones_blk, zeros):
    def body(dst_hbm, ones_hbm, zeros_hbm, out_hbm, deg_sh, dst_v, ones_v):
        c = lax.axis_index("c")
        s = lax.axis_index("s")
        _per_tile_rows(s, lambda r0, nr: pltpu.sync_copy(
            zeros_hbm.at[pl.ds(r0, nr)], deg_sh.at[pl.ds(r0, nr)]))
        pltpu.sync_copy(ones_hbm, ones_v)
        pltpu.sync_copy(dst_hbm.at[c * NS + s], dst_v)
        plsc.subcore_barrier()

        def step(j, carry):
            pltpu.sync_copy(ones_v, deg_sh.at[dst_v.at[j]], add=True)
            return carry

        lax.fori_loop(0, NBT, step, 0)
        plsc.subcore_barrier()
        _per_tile_rows(s, lambda r0, nr: pltpu.sync_copy(
            deg_sh.at[pl.ds(r0, nr)], out_hbm.at[c].at[pl.ds(r0, nr)]))

    f = pl.kernel(
        body,
        out_type=jax.ShapeDtypeStruct((NC, N, D), jnp.float32),
        mesh=_sc_mesh(),
        scratch_types=[
            pltpu.VMEM_SHARED((N, D), jnp.float32),
            pltpu.VMEM((NBT, EB), jnp.int32),
            pltpu.VMEM((EB, D), jnp.float32),
        ],
    )
    return f(dst2d, ones_blk, zeros)


def _sc_aggregate(y, src2d, dst2d, zeros):
    def body(y_hbm, src_hbm, dst_hbm, zeros_hbm, out_hbm,
             acc_sh, src_v, dst_v, rows_v, sem):
        c = lax.axis_index("c")
        s = lax.axis_index("s")

        def init(r0, nr):
            @pl.when(c == 0)
            def _():
                pltpu.sync_copy(y_hbm.at[pl.ds(r0, nr)],
                                acc_sh.at[pl.ds(r0, nr)])

            @pl.when(c != 0)
            def _():
                pltpu.sync_copy(zeros_hbm.at[pl.ds(r0, nr)],
                                acc_sh.at[pl.ds(r0, nr)])

        _per_tile_rows(s, init)
        wid = c * NS + s
        pltpu.sync_copy(src_hbm.at[wid], src_v)
        pltpu.sync_copy(dst_hbm.at[wid], dst_v)
        plsc.subcore_barrier()

        def step(j, carry):
            pltpu.async_copy(y_hbm.at[src_v.at[j]], rows_v, sem).wait()
            pltpu.sync_copy(rows_v, acc_sh.at[dst_v.at[j]], add=True)
            return carry

        lax.fori_loop(0, NBT, step, 0)
        plsc.subcore_barrier()
        _per_tile_rows(s, lambda r0, nr: pltpu.sync_copy(
            acc_sh.at[pl.ds(r0, nr)], out_hbm.at[c].at[pl.ds(r0, nr)]))

    f = pl.kernel(
        body,
        out_type=jax.ShapeDtypeStruct((NC, N, D), jnp.float32),
        mesh=_sc_mesh(),
        scratch_types=[
            pltpu.VMEM_SHARED((N, D), jnp.float32),
            pltpu.VMEM((NBT, EB), jnp.int32),
            pltpu.VMEM((NBT, EB), jnp.int32),
            pltpu.VMEM((EB, D), jnp.float32),
            pltpu.SemaphoreType.DMA,
        ],
    )
    return f(y, src2d, dst2d, zeros)


def _tc1_body(x_ref, w_ref, degp_ref, y_ref, dinv_ref):
    deg = degp_ref[0, :, 0:16] + degp_ref[1, :, 0:16] + 1.0
    dinv = lax.rsqrt(deg)
    xw = jnp.dot(x_ref[...], w_ref[...], preferred_element_type=jnp.float32)
    y_ref[...] = xw * dinv[:, 0:1]
    dinv_ref[...] = dinv


def _tc1(x, W1, degp):
    return pl.pallas_call(
        _tc1_body,
        grid=(NBR,),
        in_specs=[
            pl.BlockSpec((BR, D), lambda i: (i, 0)),
            pl.BlockSpec((D, D), lambda i: (0, 0)),
            pl.BlockSpec((NC, BR, D), lambda i: (0, i, 0)),
        ],
        out_specs=[
            pl.BlockSpec((BR, D), lambda i: (i, 0)),
            pl.BlockSpec((BR, 16), lambda i: (i, 0)),
        ],
        out_shape=[
            jax.ShapeDtypeStruct((N, D), jnp.float32),
            jax.ShapeDtypeStruct((N, 16), jnp.float32),
        ],
    )(x, W1, degp)


def _tc2_body(accp_ref, dinv_ref, b_ref, w_ref, y2_ref):
    dinv = dinv_ref[...][:, 0:1]
    h = jnp.maximum(dinv * (accp_ref[0] + accp_ref[1]) + b_ref[...], 0.0)
    y2_ref[...] = jnp.dot(h, w_ref[...],
                          preferred_element_type=jnp.float32) * dinv


def _tc2(accp, dinv, b1, W2):
    return pl.pallas_call(
        _tc2_body,
        grid=(NBR,),
        in_specs=[
            pl.BlockSpec((NC, BR, D), lambda i: (0, i, 0)),
            pl.BlockSpec((BR, 16), lambda i: (i, 0)),
            pl.BlockSpec((1, D), lambda i: (0, 0)),
            pl.BlockSpec((D, D), lambda i: (0, 0)),
        ],
        out_specs=pl.BlockSpec((BR, D), lambda i: (i, 0)),
        out_shape=jax.ShapeDtypeStruct((N, D), jnp.float32),
    )(accp, dinv, b1, W2)


def _tc3_body(accp_ref, dinv_ref, b_ref, batch_ref, fcw_ref, fcb_ref,
              out_ref, sums_ref, cnts_ref):
    i = pl.program_id(0)

    @pl.when(i == 0)
    def _():
        sums_ref[...] = jnp.zeros_like(sums_ref)
        cnts_ref[...] = jnp.zeros_like(cnts_ref)

    dinv = dinv_ref[...][:, 0:1]
    h = jnp.maximum(dinv * (accp_ref[0] + accp_ref[1]) + b_ref[...], 0.0)
    gids = lax.broadcasted_iota(jnp.int32, (BR, G), 1)
    oh = (gids == batch_ref[...]).astype(jnp.float32)
    sums_ref[...] += lax.dot_general(oh, h, (((0,), (0,)), ((), ())),
                                     preferred_element_type=jnp.float32)
    cnts_ref[...] += jnp.sum(oh, axis=0)[:, None]

    @pl.when(i == NBR - 1)
    def _():
        pooled = sums_ref[...] / jnp.maximum(cnts_ref[...], 1.0)
        out_ref[...] = jnp.dot(pooled, fcw_ref[...],
                               preferred_element_type=jnp.float32) + fcb_ref[...]


def _tc3(accp, dinv, b2, batch2d, fc_W, fc_b):
    return pl.pallas_call(
        _tc3_body,
        grid=(NBR,),
        in_specs=[
            pl.BlockSpec((NC, BR, D), lambda i: (0, i, 0)),
            pl.BlockSpec((BR, 16), lambda i: (i, 0)),
            pl.BlockSpec((1, D), lambda i: (0, 0)),
            pl.BlockSpec((BR, 1), lambda i: (i, 0)),
            pl.BlockSpec((D, D), lambda i: (0, 0)),
            pl.BlockSpec((1, D), lambda i: (0, 0)),
        ],
        out_specs=pl.BlockSpec((G, D), lambda i: (0, 0)),
        out_shape=jax.ShapeDtypeStruct((G, D), jnp.float32),
        scratch_shapes=[
            pltpu.VMEM((G, D), jnp.float32),
            pltpu.VMEM((G, 1), jnp.float32),
        ],
    )(accp, dinv, b2, batch2d, fc_W, fc_b)


def _xla_rest(x, edge_index, batch, W1, b1, W2, b2, fc_W, fc_b, degp):
    deg = degp[0, :, 0] + degp[1, :, 0]
    dinv = lax.rsqrt(deg)
    src, dst = edge_index[0], edge_index[1]
    def layer(h, W, b):
        y = (h @ W) * dinv[:, None]
        agg = jax.ops.segment_sum(y[src], dst, num_segments=N) + y
        return jnp.maximum(dinv[:, None] * agg + b, 0.0)
    h = layer(x, W1, b1)
    h = layer(h, W2, b2)
    sums = jax.ops.segment_sum(h, batch, num_segments=G)
    counts = jax.ops.segment_sum(jnp.ones((N,), h.dtype), batch, num_segments=G)
    pooled = sums / jnp.maximum(counts, 1.0)[:, None]
    return pooled @ fc_W + fc_b


def kernel(x, edge_index, batch, W1, b1, W2, b2, fc_W, fc_b):
    src2d = edge_index[0].reshape(NW, NBT, EB)
    dst2d = edge_index[1].reshape(NW, NBT, EB)
    zeros = jnp.zeros((N, D), jnp.float32)
    ones_blk = jnp.ones((EB, D), jnp.float32)

    degp = _sc_degree(dst2d, ones_blk, zeros)
    y1, dinv = _tc1(x, W1, degp)
    acc1 = _sc_aggregate(y1, src2d, dst2d, zeros)
    y2 = _tc2(acc1, dinv, b1.reshape(1, D), W2)
    acc2 = _sc_aggregate(y2, src2d, dst2d, zeros)
    return _tc3(acc2, dinv, b2.reshape(1, D), batch.reshape(N, 1),
                fc_W, fc_b.reshape(1, D))
